# baseline (speedup 1.0000x reference)
import functools

import jax
import jax.numpy as jnp
from jax import lax
from jax.experimental import pallas as pl
from jax.experimental.pallas import tpu as pltpu
from jax.experimental.pallas import tpu_sc as plsc

VOCAB = 100000
EMBED = 64
CONTEXT = 200
HIDDEN = 128

CHUNK = 8192
NCHUNK = -(-VOCAB // CHUNK)
VPAD = NCHUNK * CHUNK

NC, NS = 2, 16
NW = NC * NS
B_PAD = 256
B_PER_W = B_PAD // NW


@functools.cache
def _gather_sc():
    @functools.partial(
        pl.kernel,
        mesh=plsc.VectorSubcoreMesh(core_axis_name="c", subcore_axis_name="s"),
        out_type=jax.ShapeDtypeStruct((B_PAD, EMBED), jnp.float32),
        scratch_types=[
            pltpu.VMEM((B_PER_W,), jnp.int32),
            pltpu.VMEM((B_PER_W, EMBED), jnp.float32),
            pltpu.SemaphoreType.DMA,
        ],
        compiler_params=pltpu.CompilerParams(use_tc_tiling_on_sc=False),
    )
    def gather(table_hbm, idx_hbm, out_hbm, idx_v, rows_v, sem):
        wid = lax.axis_index("s") * NC + lax.axis_index("c")
        base = wid * B_PER_W
        pltpu.sync_copy(idx_hbm.at[pl.ds(base, B_PER_W)], idx_v)
        pltpu.async_copy(table_hbm.at[idx_v], rows_v, sem).wait()
        pltpu.sync_copy(rows_v, out_hbm.at[pl.ds(base, B_PER_W)])

    return gather


def _dense_body(e_ref, w1_ref, b1_ref, w2_ref, b2_ref, o_ref, h_ref):
    k = pl.program_id(0)

    @pl.when(k == 0)
    def _():
        h = jnp.dot(e_ref[...], w1_ref[...], preferred_element_type=jnp.float32)
        h_ref[...] = jnp.maximum(h + b1_ref[...], 0.0)

    chunk = jnp.dot(h_ref[...], w2_ref[...], preferred_element_type=jnp.float32)
    chunk = chunk + b2_ref[...]
    col = k * CHUNK + lax.broadcasted_iota(jnp.int32, (1, CHUNK), 1)
    chunk = jnp.where(col < VOCAB, chunk, -1e30)
    o_ref[:, pl.ds(k * CHUNK, CHUNK)] = chunk

    @pl.when(k == NCHUNK - 1)
    def _():
        logits = o_ref[...]
        m = jnp.max(logits)
        lse = m + jnp.log(jnp.sum(jnp.exp(logits - m)))
        o_ref[...] = logits - lse


def _dense_call(e, W1, b1r, W2, b2r, interpret=False):
    return pl.pallas_call(
        _dense_body,
        grid=(NCHUNK,),
        in_specs=[
            pl.BlockSpec((1, CONTEXT * EMBED), lambda k: (0, 0)),
            pl.BlockSpec((CONTEXT * EMBED, HIDDEN), lambda k: (0, 0)),
            pl.BlockSpec((1, HIDDEN), lambda k: (0, 0)),
            pl.BlockSpec((HIDDEN, CHUNK), lambda k: (0, k)),
            pl.BlockSpec((1, CHUNK), lambda k: (0, k)),
        ],
        out_specs=pl.BlockSpec((1, VPAD), lambda k: (0, 0)),
        out_shape=jax.ShapeDtypeStruct((1, VPAD), jnp.float32),
        scratch_shapes=[pltpu.VMEM((1, HIDDEN), jnp.float32)],
        interpret=interpret,
    )(e, W1, b1r, W2, b2r)


def kernel(inputs, embeddings, W1, b1, W2, b2):
    idx = jnp.zeros((B_PAD,), jnp.int32).at[:CONTEXT].set(inputs.astype(jnp.int32))
    rows = _gather_sc()(embeddings, idx)
    e = rows[:CONTEXT].reshape(1, CONTEXT * EMBED)
    out = _dense_call(e, W1, b1.reshape(1, HIDDEN), W2, b2.reshape(1, VOCAB))
    return out[:, :VOCAB]


# speedup vs baseline: 1.2309x; 1.2309x over previous
import functools

import jax
import jax.numpy as jnp
from jax import lax
from jax.experimental import pallas as pl
from jax.experimental.pallas import tpu as pltpu
from jax.experimental.pallas import tpu_sc as plsc

VOCAB = 100000
EMBED = 64
CONTEXT = 200
HIDDEN = 128

CHUNK = 8192
NCHUNK = -(-VOCAB // CHUNK)
VPAD = NCHUNK * CHUNK

NC, NS = 2, 16
NW = NC * NS
LANES = 16
B_PAD = NW * LANES
B_PER_W = B_PAD // NW
SUPER = 12500


@functools.cache
def _gather_sc():
    @functools.partial(
        pl.kernel,
        mesh=plsc.VectorSubcoreMesh(core_axis_name="c", subcore_axis_name="s"),
        out_type=jax.ShapeDtypeStruct((B_PAD, EMBED), jnp.float32),
        scratch_types=[
            pltpu.VMEM((B_PER_W,), jnp.int32),
            pltpu.VMEM((B_PER_W, EMBED), jnp.float32),
            pltpu.SemaphoreType.DMA,
        ],
        compiler_params=pltpu.CompilerParams(needs_layout_passes=False),
    )
    def gather(table_hbm, idx_hbm, out_hbm, idx_v, out_v, sem):
        wid = lax.axis_index("s") * NC + lax.axis_index("c")
        base = wid * B_PER_W
        pltpu.sync_copy(idx_hbm.at[pl.ds(base, B_PER_W)], idx_v)
        iv = idx_v[...]
        sidx = lax.shift_right_logical(iv, 3)
        sub = lax.bitwise_and(iv, 7)
        t = lax.iota(jnp.int32, LANES)
        copies = []
        for i in range(B_PER_W):
            s_i = jnp.max(jnp.where(t == i, sidx, 0))
            r_i = jnp.max(jnp.where(t == i, sub, 0))
            copies.append(pltpu.async_copy(table_hbm.at[s_i, r_i], out_v.at[i], sem))
        for c in copies:
            c.wait()
        pltpu.sync_copy(out_v, out_hbm.at[pl.ds(base, B_PER_W)])

    return gather


def _dense_body(e_ref, w1_ref, b1_ref, w2_ref, b2_ref, o_ref, h_ref):
    k = pl.program_id(0)

    @pl.when(k == 0)
    def _():
        h = jnp.dot(e_ref[...], w1_ref[...], preferred_element_type=jnp.float32)
        h_ref[...] = jnp.maximum(h + b1_ref[...], 0.0)

    chunk = jnp.dot(h_ref[...], w2_ref[...], preferred_element_type=jnp.float32)
    chunk = chunk + b2_ref[...]
    col = k * CHUNK + lax.broadcasted_iota(jnp.int32, (1, CHUNK), 1)
    chunk = jnp.where(col < VOCAB, chunk, -1e30)
    o_ref[:, pl.ds(k * CHUNK, CHUNK)] = chunk

    @pl.when(k == NCHUNK - 1)
    def _():
        logits = o_ref[...]
        m = jnp.max(logits)
        lse = m + jnp.log(jnp.sum(jnp.exp(logits - m)))
        o_ref[...] = logits - lse


def _dense_call(e, W1, b1r, W2, b2r, interpret=False):
    return pl.pallas_call(
        _dense_body,
        grid=(NCHUNK,),
        in_specs=[
            pl.BlockSpec((1, CONTEXT * EMBED), lambda k: (0, 0)),
            pl.BlockSpec((CONTEXT * EMBED, HIDDEN), lambda k: (0, 0)),
            pl.BlockSpec((1, HIDDEN), lambda k: (0, 0)),
            pl.BlockSpec((HIDDEN, CHUNK), lambda k: (0, k)),
            pl.BlockSpec((1, CHUNK), lambda k: (0, k)),
        ],
        out_specs=pl.BlockSpec((1, VPAD), lambda k: (0, 0)),
        out_shape=jax.ShapeDtypeStruct((1, VPAD), jnp.float32),
        scratch_shapes=[pltpu.VMEM((1, HIDDEN), jnp.float32)],
        interpret=interpret,
    )(e, W1, b1r, W2, b2r)


def kernel(inputs, embeddings, W1, b1, W2, b2):
    idx = jnp.zeros((B_PAD,), jnp.int32).at[:CONTEXT].set(inputs.astype(jnp.int32))
    table3 = embeddings.reshape(SUPER, 8, EMBED)
    rows = _gather_sc()(table3, idx)
    e = rows[:CONTEXT].reshape(1, CONTEXT * EMBED)
    out = _dense_call(e, W1, b1.reshape(1, HIDDEN), W2, b2.reshape(1, VOCAB))
    return out[:, :VOCAB]
